# baseline (speedup 1.0000x reference)
import jax
import jax.numpy as jnp
from jax.experimental import pallas as pl
from jax.experimental.pallas import tpu as pltpu

SMOOTH = 0.1
BS, C, H, W = 4, 96, 384, 384
HB = 8
GB, GH = BS, H // HB
NPIX = BS * H * W
NUMEL = BS * C * H * W


def _dice_body(yh_ref, y_ref, o_ref, acc_ref):
    b = pl.program_id(0)
    h = pl.program_id(1)
    is_first = jnp.logical_and(b == 0, h == 0)
    is_last = jnp.logical_and(b == GB - 1, h == GH - 1)

    @pl.when(is_first)
    def _():
        acc_ref[...] = jnp.zeros_like(acc_ref)

    lbl = y_ref[0]
    a_t = None
    a_i = jnp.zeros((HB, W), jnp.float32)
    for ci in range(C):
        t = jnp.tanh(yh_ref[0, ci] * 0.5)
        a_i = jnp.where(lbl == ci, t, a_i)
        a_t = t if a_t is None else a_t + t
    acc_ref[0] += a_t
    acc_ref[1] += a_i

    @pl.when(is_last)
    def _():
        t_sum = jnp.sum(acc_ref[0])
        i_sum = jnp.sum(acc_ref[1])
        p_sum = 0.5 * t_sum + 0.5 * NUMEL
        inter = 0.5 * i_sum + 0.5 * NPIX
        o_ref[0] = 1.0 - (2.0 * inter + SMOOTH) / (NPIX + p_sum + SMOOTH)


@jax.jit
def kernel(y_hat, y):
    out = pl.pallas_call(
        _dice_body,
        grid=(GB, GH),
        in_specs=[
            pl.BlockSpec((1, C, HB, W), lambda b, h: (b, 0, h, 0)),
            pl.BlockSpec((1, HB, W), lambda b, h: (b, h, 0)),
        ],
        out_specs=pl.BlockSpec(
            (1,), lambda b, h: (0,), memory_space=pltpu.MemorySpace.SMEM),
        out_shape=jax.ShapeDtypeStruct((1,), jnp.float32),
        scratch_shapes=[pltpu.VMEM((2, HB, W), jnp.float32)],
    )(y_hat, y)
    return out[0]


# speedup vs baseline: 1.0083x; 1.0083x over previous
import jax
import jax.numpy as jnp
from jax.experimental import pallas as pl
from jax.experimental.pallas import tpu as pltpu

SMOOTH = 0.1
BS, C, H, W = 4, 96, 384, 384
CB = 2
SH = 8
NS = H // SH
GB, GC = BS, C // CB
NPIX = BS * H * W
NUMEL = BS * C * H * W


def _dice_body(yh_ref, y_ref, o_ref, acc_ref):
    b = pl.program_id(0)
    c = pl.program_id(1)
    is_first = jnp.logical_and(b == 0, c == 0)
    is_last = jnp.logical_and(b == GB - 1, c == GC - 1)

    @pl.when(is_first)
    def _():
        acc_ref[...] = jnp.zeros_like(acc_ref)

    for s in range(NS):
        lbl = y_ref[0, s * SH:(s + 1) * SH, :]
        a_t = None
        a_i = None
        for ci in range(CB):
            t = jnp.tanh(yh_ref[0, ci, s * SH:(s + 1) * SH, :] * 0.5)
            ti = jnp.where(lbl == c * CB + ci, t, 0.0)
            a_t = t if a_t is None else a_t + t
            a_i = ti if a_i is None else a_i + ti
        acc_ref[0] += a_t
        acc_ref[1] += a_i

    @pl.when(is_last)
    def _():
        t_sum = jnp.sum(acc_ref[0])
        i_sum = jnp.sum(acc_ref[1])
        p_sum = 0.5 * t_sum + 0.5 * NUMEL
        inter = 0.5 * i_sum + 0.5 * NPIX
        o_ref[0] = 1.0 - (2.0 * inter + SMOOTH) / (NPIX + p_sum + SMOOTH)


@jax.jit
def kernel(y_hat, y):
    out = pl.pallas_call(
        _dice_body,
        grid=(GB, GC),
        in_specs=[
            pl.BlockSpec((1, CB, H, W), lambda b, c: (b, c, 0, 0)),
            pl.BlockSpec((1, H, W), lambda b, c: (b, 0, 0)),
        ],
        out_specs=pl.BlockSpec(
            (1,), lambda b, c: (0,), memory_space=pltpu.MemorySpace.SMEM),
        out_shape=jax.ShapeDtypeStruct((1,), jnp.float32),
        scratch_shapes=[pltpu.VMEM((2, SH, W), jnp.float32)],
    )(y_hat, y)
    return out[0]


# speedup vs baseline: 2.2210x; 2.2026x over previous
import jax
import jax.numpy as jnp
from jax.experimental import pallas as pl
from jax.experimental.pallas import tpu as pltpu

SMOOTH = 0.1
BS, C, H, W = 4, 96, 384, 384
CB = 16
SH = 8
NS = H // SH
GB, GC = BS, C // CB
NPIX = BS * H * W
NUMEL = BS * C * H * W


def _dice_body(yh_ref, y_ref, o_ref, acc_ref):
    b = pl.program_id(0)
    c = pl.program_id(1)
    is_first = jnp.logical_and(b == 0, c == 0)
    is_last = jnp.logical_and(b == GB - 1, c == GC - 1)

    @pl.when(is_first)
    def _():
        acc_ref[...] = jnp.zeros_like(acc_ref)

    for s in range(NS):
        lbl = y_ref[0, s * SH:(s + 1) * SH, :]
        a_t = None
        a_i = None
        for ci in range(CB):
            t = jnp.tanh(yh_ref[0, ci, s * SH:(s + 1) * SH, :] * 0.5)
            ti = jnp.where(lbl == c * CB + ci, t, 0.0)
            a_t = t if a_t is None else a_t + t
            a_i = ti if a_i is None else a_i + ti
        acc_ref[0] += a_t
        acc_ref[1] += a_i

    @pl.when(is_last)
    def _():
        t_sum = jnp.sum(acc_ref[0])
        i_sum = jnp.sum(acc_ref[1])
        p_sum = 0.5 * t_sum + 0.5 * NUMEL
        inter = 0.5 * i_sum + 0.5 * NPIX
        o_ref[0] = 1.0 - (2.0 * inter + SMOOTH) / (NPIX + p_sum + SMOOTH)


@jax.jit
def kernel(y_hat, y):
    out = pl.pallas_call(
        _dice_body,
        grid=(GB, GC),
        in_specs=[
            pl.BlockSpec((1, CB, H, W), lambda b, c: (b, c, 0, 0)),
            pl.BlockSpec((1, H, W), lambda b, c: (b, 0, 0)),
        ],
        out_specs=pl.BlockSpec(
            (1,), lambda b, c: (0,), memory_space=pltpu.MemorySpace.SMEM),
        out_shape=jax.ShapeDtypeStruct((1,), jnp.float32),
        scratch_shapes=[pltpu.VMEM((2, SH, W), jnp.float32)],
    )(y_hat, y)
    return out[0]


# speedup vs baseline: 2.2421x; 1.0095x over previous
import jax
import jax.numpy as jnp
from jax.experimental import pallas as pl
from jax.experimental.pallas import tpu as pltpu

SMOOTH = 0.1
BS, C, H, W = 4, 96, 384, 384
CB = 32
SH = 8
NS = H // SH
GB, GC = BS, C // CB
NPIX = BS * H * W
NUMEL = BS * C * H * W


def _dice_body(yh_ref, y_ref, o_ref, acc_ref):
    b = pl.program_id(0)
    c = pl.program_id(1)
    is_first = jnp.logical_and(b == 0, c == 0)
    is_last = jnp.logical_and(b == GB - 1, c == GC - 1)

    @pl.when(is_first)
    def _():
        acc_ref[...] = jnp.zeros_like(acc_ref)

    for s in range(NS):
        lbl = y_ref[0, s * SH:(s + 1) * SH, :]
        a_t = None
        a_i = None
        for ci in range(CB):
            t = jnp.tanh(yh_ref[0, ci, s * SH:(s + 1) * SH, :] * 0.5)
            ti = jnp.where(lbl == c * CB + ci, t, 0.0)
            a_t = t if a_t is None else a_t + t
            a_i = ti if a_i is None else a_i + ti
        acc_ref[0] += a_t
        acc_ref[1] += a_i

    @pl.when(is_last)
    def _():
        t_sum = jnp.sum(acc_ref[0])
        i_sum = jnp.sum(acc_ref[1])
        p_sum = 0.5 * t_sum + 0.5 * NUMEL
        inter = 0.5 * i_sum + 0.5 * NPIX
        o_ref[0] = 1.0 - (2.0 * inter + SMOOTH) / (NPIX + p_sum + SMOOTH)


@jax.jit
def kernel(y_hat, y):
    out = pl.pallas_call(
        _dice_body,
        grid=(GB, GC),
        in_specs=[
            pl.BlockSpec((1, CB, H, W), lambda b, c: (b, c, 0, 0)),
            pl.BlockSpec((1, H, W), lambda b, c: (b, 0, 0)),
        ],
        out_specs=pl.BlockSpec(
            (1,), lambda b, c: (0,), memory_space=pltpu.MemorySpace.SMEM),
        out_shape=jax.ShapeDtypeStruct((1,), jnp.float32),
        scratch_shapes=[pltpu.VMEM((2, SH, W), jnp.float32)],
    )(y_hat, y)
    return out[0]


# speedup vs baseline: 2.2495x; 1.0033x over previous
import jax
import jax.numpy as jnp
from jax.experimental import pallas as pl
from jax.experimental.pallas import tpu as pltpu

SMOOTH = 0.1
BS, C, H, W = 4, 96, 384, 384
CB = 32
SH = 8
NS = H // SH
GB, GC = BS, C // CB
NPIX = BS * H * W
NUMEL = BS * C * H * W


def _dice_body(yh_ref, y_ref, o_ref, acc_ref):
    b = pl.program_id(0)
    c = pl.program_id(1)
    is_first = jnp.logical_and(b == 0, c == 0)
    is_last = jnp.logical_and(b == GB - 1, c == GC - 1)

    @pl.when(is_first)
    def _():
        acc_ref[...] = jnp.zeros_like(acc_ref)

    for s in range(NS):
        lbl = y_ref[0, s * SH:(s + 1) * SH, :]
        a_t = None
        a_i = None
        for ci in range(CB):
            t = jnp.tanh(yh_ref[0, ci, s * SH:(s + 1) * SH, :] * 0.5)
            m = lbl == c * CB + ci
            a_t = t if a_t is None else a_t + t
            a_i = jnp.where(m, t, 0.0 if a_i is None else a_i)
        acc_ref[0] += a_t
        acc_ref[1] += a_i

    @pl.when(is_last)
    def _():
        t_sum = jnp.sum(acc_ref[0])
        i_sum = jnp.sum(acc_ref[1])
        p_sum = 0.5 * t_sum + 0.5 * NUMEL
        inter = 0.5 * i_sum + 0.5 * NPIX
        o_ref[0] = 1.0 - (2.0 * inter + SMOOTH) / (NPIX + p_sum + SMOOTH)


@jax.jit
def kernel(y_hat, y):
    out = pl.pallas_call(
        _dice_body,
        grid=(GB, GC),
        in_specs=[
            pl.BlockSpec((1, CB, H, W), lambda b, c: (b, c, 0, 0)),
            pl.BlockSpec((1, H, W), lambda b, c: (b, 0, 0)),
        ],
        out_specs=pl.BlockSpec(
            (1,), lambda b, c: (0,), memory_space=pltpu.MemorySpace.SMEM),
        out_shape=jax.ShapeDtypeStruct((1,), jnp.float32),
        scratch_shapes=[pltpu.VMEM((2, SH, W), jnp.float32)],
    )(y_hat, y)
    return out[0]
